# baseline (device time: 117924 ns/iter reference)
import jax
import jax.numpy as jnp
from jax import lax
from jax.experimental import pallas as pl
from jax.experimental.pallas import tpu as pltpu

N_DEV = 32
CHUNK = 1024 // N_DEV


def kernel(x, w_mat):
    m, _ = x.shape
    _, n = w_mat.shape

    def body(x_ref, w_ref, out_ref, rs_buf,
             rs_send_sems, rs_recv_sems, ag_send_sems, ag_recv_sems):
        my = lax.axis_index("i")

        out_ref[:, :] = jnp.dot(
            x_ref[:, :], w_ref[:, :], preferred_element_type=jnp.float32
        )

        barrier_sem = pltpu.get_barrier_semaphore()
        for o in range(1, N_DEV):
            peer = jnp.mod(my + o, N_DEV)
            pl.semaphore_signal(
                barrier_sem, inc=1, device_id=(peer,),
                device_id_type=pl.DeviceIdType.MESH,
            )
        pl.semaphore_wait(barrier_sem, N_DEV - 1)

        rs = []
        for o in range(1, N_DEV):
            tgt = jnp.mod(my + o, N_DEV)
            rdma = pltpu.make_async_remote_copy(
                src_ref=out_ref.at[pl.ds(tgt * CHUNK, CHUNK)],
                dst_ref=rs_buf.at[o - 1],
                send_sem=rs_send_sems.at[o - 1],
                recv_sem=rs_recv_sems.at[o - 1],
                device_id=(tgt,),
                device_id_type=pl.DeviceIdType.MESH,
            )
            rdma.start()
            rs.append(rdma)

        for r in rs:
            r.wait_recv()
        acc = out_ref[pl.ds(my * CHUNK, CHUNK), :] + jnp.sum(
            rs_buf[:, :, :], axis=0
        )
        out_ref[pl.ds(my * CHUNK, CHUNK), :] = acc

        ag = []
        for o in range(1, N_DEV):
            tgt = jnp.mod(my + o, N_DEV)
            rdma = pltpu.make_async_remote_copy(
                src_ref=out_ref.at[pl.ds(my * CHUNK, CHUNK)],
                dst_ref=out_ref.at[pl.ds(my * CHUNK, CHUNK)],
                send_sem=ag_send_sems.at[o - 1],
                recv_sem=ag_recv_sems.at[o - 1],
                device_id=(tgt,),
                device_id_type=pl.DeviceIdType.MESH,
            )
            rdma.start()
            ag.append(rdma)

        for r in ag:
            r.wait_recv()
        for r in rs:
            r.wait_send()
        for r in ag:
            r.wait_send()

    return pl.pallas_call(
        body,
        out_shape=jax.ShapeDtypeStruct((m, n), jnp.float32),
        in_specs=[
            pl.BlockSpec(memory_space=pltpu.VMEM),
            pl.BlockSpec(memory_space=pltpu.VMEM),
        ],
        out_specs=pl.BlockSpec(memory_space=pltpu.VMEM),
        scratch_shapes=[
            pltpu.VMEM((N_DEV - 1, CHUNK, n), jnp.float32),
            pltpu.SemaphoreType.DMA((N_DEV - 1,)),
            pltpu.SemaphoreType.DMA((N_DEV - 1,)),
            pltpu.SemaphoreType.DMA((N_DEV - 1,)),
            pltpu.SemaphoreType.DMA((N_DEV - 1,)),
        ],
        compiler_params=pltpu.CompilerParams(collective_id=0),
    )(x, w_mat)


# device time: 110815 ns/iter; 1.0642x vs baseline; 1.0642x over previous
import jax
import jax.numpy as jnp
from jax import lax
from jax.experimental import pallas as pl
from jax.experimental.pallas import tpu as pltpu

N_DEV = 32
CHUNK = 1024 // N_DEV
G = 2


def kernel(x, w_mat):
    m, _ = x.shape
    _, n = w_mat.shape
    ng = n // G

    def body(x_ref, w_ref, out_ref, rs_buf,
             rs_send_sems, rs_recv_sems, ag_send_sems, ag_recv_sems):
        my = lax.axis_index("i")

        barrier_sem = pltpu.get_barrier_semaphore()
        for o in range(1, N_DEV):
            peer = jnp.mod(my + o, N_DEV)
            pl.semaphore_signal(
                barrier_sem, inc=1, device_id=(peer,),
                device_id_type=pl.DeviceIdType.MESH,
            )
        pl.semaphore_wait(barrier_sem, N_DEV - 1)

        rs = [[None] * (N_DEV - 1) for _ in range(G)]
        ag = [[None] * (N_DEV - 1) for _ in range(G)]

        def start_rs(g):
            c0 = g * ng
            out_ref[:, pl.ds(c0, ng)] = jnp.dot(
                x_ref[:, :], w_ref[:, pl.ds(c0, ng)],
                preferred_element_type=jnp.float32,
            )
            for o in range(1, N_DEV):
                tgt = jnp.mod(my + o, N_DEV)
                rdma = pltpu.make_async_remote_copy(
                    src_ref=out_ref.at[pl.ds(tgt * CHUNK, CHUNK),
                                       pl.ds(c0, ng)],
                    dst_ref=rs_buf.at[g, o - 1],
                    send_sem=rs_send_sems.at[g, o - 1],
                    recv_sem=rs_recv_sems.at[g, o - 1],
                    device_id=(tgt,),
                    device_id_type=pl.DeviceIdType.MESH,
                )
                rdma.start()
                rs[g][o - 1] = rdma

        def finish_rs_start_ag(g):
            c0 = g * ng
            for r in rs[g]:
                r.wait_recv()
            acc = out_ref[pl.ds(my * CHUNK, CHUNK), pl.ds(c0, ng)] + jnp.sum(
                rs_buf[g], axis=0
            )
            out_ref[pl.ds(my * CHUNK, CHUNK), pl.ds(c0, ng)] = acc
            for o in range(1, N_DEV):
                tgt = jnp.mod(my + o, N_DEV)
                rdma = pltpu.make_async_remote_copy(
                    src_ref=out_ref.at[pl.ds(my * CHUNK, CHUNK),
                                       pl.ds(c0, ng)],
                    dst_ref=out_ref.at[pl.ds(my * CHUNK, CHUNK),
                                       pl.ds(c0, ng)],
                    send_sem=ag_send_sems.at[g, o - 1],
                    recv_sem=ag_recv_sems.at[g, o - 1],
                    device_id=(tgt,),
                    device_id_type=pl.DeviceIdType.MESH,
                )
                rdma.start()
                ag[g][o - 1] = rdma

        start_rs(0)
        for g in range(G):
            if g + 1 < G:
                start_rs(g + 1)
            finish_rs_start_ag(g)

        for g in range(G):
            for r in ag[g]:
                r.wait_recv()
            for r in rs[g]:
                r.wait_send()
            for r in ag[g]:
                r.wait_send()

    return pl.pallas_call(
        body,
        out_shape=jax.ShapeDtypeStruct((m, n), jnp.float32),
        in_specs=[
            pl.BlockSpec(memory_space=pltpu.VMEM),
            pl.BlockSpec(memory_space=pltpu.VMEM),
        ],
        out_specs=pl.BlockSpec(memory_space=pltpu.VMEM),
        scratch_shapes=[
            pltpu.VMEM((G, N_DEV - 1, CHUNK, ng), jnp.float32),
            pltpu.SemaphoreType.DMA((G, N_DEV - 1)),
            pltpu.SemaphoreType.DMA((G, N_DEV - 1)),
            pltpu.SemaphoreType.DMA((G, N_DEV - 1)),
            pltpu.SemaphoreType.DMA((G, N_DEV - 1)),
        ],
        compiler_params=pltpu.CompilerParams(collective_id=0),
    )(x, w_mat)


# device time: 83821 ns/iter; 1.4069x vs baseline; 1.3220x over previous
import jax
import jax.numpy as jnp
from jax import lax
from jax.experimental import pallas as pl
from jax.experimental.pallas import tpu as pltpu

N_DEV = 32
N_CLS = N_DEV // 2
CHUNK = 1024 // N_DEV
G = 4


def kernel(x, w_mat):
    m, _ = x.shape
    _, n = w_mat.shape
    ng = n // G
    half = m // 2

    def body(x_ref, w_ref, out_ref, rx_buf, rs_buf,
             r1_ssem, r1_rsem, rs_ssem, rs_rsem,
             ag_ssem, ag_rsem, x2_ssem, x2_rsem):
        my = lax.axis_index("i")
        q = jnp.mod(my, 2)
        partner = my + 1 - 2 * q
        my_half = q * half
        other_half = (1 - q) * half
        my_row0 = my_half + (my // 2) * CHUNK

        barrier_sem = pltpu.get_barrier_semaphore()
        pl.semaphore_signal(
            barrier_sem, inc=1, device_id=(partner,),
            device_id_type=pl.DeviceIdType.MESH,
        )
        for j in range(1, N_CLS):
            peer = jnp.mod(my + 2 * j, N_DEV)
            pl.semaphore_signal(
                barrier_sem, inc=1, device_id=(peer,),
                device_id_type=pl.DeviceIdType.MESH,
            )
        pl.semaphore_wait(barrier_sem, N_CLS)

        r1 = [None] * G
        rs = [[None] * (N_CLS - 1) for _ in range(G)]
        ag = [[None] * (N_CLS - 1) for _ in range(G)]
        x2 = [None] * G

        def start_r1(g):
            c0 = g * ng
            out_ref[:, pl.ds(c0, ng)] = jnp.dot(
                x_ref[:, :], w_ref[:, pl.ds(c0, ng)],
                preferred_element_type=jnp.float32,
            )
            rdma = pltpu.make_async_remote_copy(
                src_ref=out_ref.at[pl.ds(other_half, half), pl.ds(c0, ng)],
                dst_ref=rx_buf.at[g],
                send_sem=r1_ssem.at[g],
                recv_sem=r1_rsem.at[g],
                device_id=(partner,),
                device_id_type=pl.DeviceIdType.MESH,
            )
            rdma.start()
            r1[g] = rdma

        def start_r2(g):
            c0 = g * ng
            r1[g].wait_recv()
            out_ref[pl.ds(my_half, half), pl.ds(c0, ng)] = (
                out_ref[pl.ds(my_half, half), pl.ds(c0, ng)] + rx_buf[g]
            )
            for j in range(1, N_CLS):
                tgt = jnp.mod(my + 2 * j, N_DEV)
                tgt_row0 = my_half + (tgt // 2) * CHUNK
                rdma = pltpu.make_async_remote_copy(
                    src_ref=out_ref.at[pl.ds(tgt_row0, CHUNK), pl.ds(c0, ng)],
                    dst_ref=rs_buf.at[g, j - 1],
                    send_sem=rs_ssem.at[g, j - 1],
                    recv_sem=rs_rsem.at[g, j - 1],
                    device_id=(tgt,),
                    device_id_type=pl.DeviceIdType.MESH,
                )
                rdma.start()
                rs[g][j - 1] = rdma

        def start_ag1(g):
            c0 = g * ng
            for r in rs[g]:
                r.wait_recv()
            acc = out_ref[pl.ds(my_row0, CHUNK), pl.ds(c0, ng)] + jnp.sum(
                rs_buf[g], axis=0
            )
            out_ref[pl.ds(my_row0, CHUNK), pl.ds(c0, ng)] = acc
            for j in range(1, N_CLS):
                tgt = jnp.mod(my + 2 * j, N_DEV)
                rdma = pltpu.make_async_remote_copy(
                    src_ref=out_ref.at[pl.ds(my_row0, CHUNK), pl.ds(c0, ng)],
                    dst_ref=out_ref.at[pl.ds(my_row0, CHUNK), pl.ds(c0, ng)],
                    send_sem=ag_ssem.at[g, j - 1],
                    recv_sem=ag_rsem.at[g, j - 1],
                    device_id=(tgt,),
                    device_id_type=pl.DeviceIdType.MESH,
                )
                rdma.start()
                ag[g][j - 1] = rdma

        def start_x2(g):
            c0 = g * ng
            for r in ag[g]:
                r.wait_recv()
            rdma = pltpu.make_async_remote_copy(
                src_ref=out_ref.at[pl.ds(my_half, half), pl.ds(c0, ng)],
                dst_ref=out_ref.at[pl.ds(my_half, half), pl.ds(c0, ng)],
                send_sem=x2_ssem.at[g],
                recv_sem=x2_rsem.at[g],
                device_id=(partner,),
                device_id_type=pl.DeviceIdType.MESH,
            )
            rdma.start()
            x2[g] = rdma

        stages = [start_r1, start_r2, start_ag1, start_x2]
        for step in range(G + len(stages) - 1):
            for s, fn in enumerate(stages):
                g = step - s
                if 0 <= g < G:
                    fn(g)

        for g in range(G):
            x2[g].wait_recv()
            r1[g].wait_send()
            x2[g].wait_send()
            for r in rs[g]:
                r.wait_send()
            for r in ag[g]:
                r.wait_send()

    return pl.pallas_call(
        body,
        out_shape=jax.ShapeDtypeStruct((m, n), jnp.float32),
        in_specs=[
            pl.BlockSpec(memory_space=pltpu.VMEM),
            pl.BlockSpec(memory_space=pltpu.VMEM),
        ],
        out_specs=pl.BlockSpec(memory_space=pltpu.VMEM),
        scratch_shapes=[
            pltpu.VMEM((G, half, ng), jnp.float32),
            pltpu.VMEM((G, N_CLS - 1, CHUNK, ng), jnp.float32),
            pltpu.SemaphoreType.DMA((G,)),
            pltpu.SemaphoreType.DMA((G,)),
            pltpu.SemaphoreType.DMA((G, N_CLS - 1)),
            pltpu.SemaphoreType.DMA((G, N_CLS - 1)),
            pltpu.SemaphoreType.DMA((G, N_CLS - 1)),
            pltpu.SemaphoreType.DMA((G, N_CLS - 1)),
            pltpu.SemaphoreType.DMA((G,)),
            pltpu.SemaphoreType.DMA((G,)),
        ],
        compiler_params=pltpu.CompilerParams(collective_id=0),
    )(x, w_mat)
